# baseline (speedup 1.0000x reference)
import functools

import jax
import jax.numpy as jnp
from jax import lax
from jax.experimental import pallas as pl
from jax.experimental.pallas import tpu as pltpu
from jax.experimental.pallas import tpu_sc as plsc

_B = 8
_N = 4096
_S = 1024
_K = 32
_C = 64
_RADIUS = 0.2
_SBLK = 256
_NW = 32
_GPW = (_B * _S) // _NW
_WPB = 16
_NWORD = _N // _WPB
_CB = 8
_NB = _GPW // _CB
_TRASH = _GPW * _K


def _fps_body(xyzt_ref, cxyz_ref):
    x0 = xyzt_ref[0]
    x1 = xyzt_ref[1]
    x2 = xyzt_ref[2]
    lane = lax.broadcasted_iota(jnp.int32, (_B, _N), 1)

    def body(i, carry):
        dist, far = carry
        oh = (lane == far).astype(jnp.float32)
        cx = jnp.sum(x0 * oh, axis=1, keepdims=True)
        cy = jnp.sum(x1 * oh, axis=1, keepdims=True)
        cz = jnp.sum(x2 * oh, axis=1, keepdims=True)
        cxyz_ref[:, pl.ds(i, 1), :] = jnp.concatenate(
            [cx.T[None], cy.T[None], cz.T[None]], axis=0)
        d0 = x0 - cx
        d1 = x1 - cy
        d2 = x2 - cz
        d = d0 * d0 + d1 * d1 + d2 * d2
        dist = jnp.minimum(dist, d)
        m = jnp.max(dist, axis=1, keepdims=True)
        far = jnp.min(jnp.where(dist == m, lane, _N), axis=1,
                      keepdims=True).astype(jnp.int32)
        return dist, far

    dist0 = jnp.full((_B, _N), 1e10, jnp.float32)
    far0 = jnp.zeros((_B, 1), jnp.int32)
    lax.fori_loop(0, _S, body, (dist0, far0))


def _fps(xyzt):
    return pl.pallas_call(
        _fps_body,
        out_shape=jax.ShapeDtypeStruct((3, _S, _B), jnp.float32),
    )(xyzt)


def _bq_body(nx_ref, xyzt_ref, xyz_ref, featT_ref, w1xt_ref, w1ft_ref, b1_ref,
             p_ref, pc_ref, words_ref, qb_ref, t_ref, nw_ref):
    s = pl.program_id(1)
    nx = nx_ref[0]
    xyzt = xyzt_ref[0]
    c2 = jnp.sum(nx * nx, axis=1, keepdims=True)
    p2 = jnp.sum(xyzt * xyzt, axis=0, keepdims=True)
    dot = jnp.dot(nx, xyzt, preferred_element_type=jnp.float32)
    sqr = c2 + p2 - 2.0 * dot
    maskf = jnp.where(sqr > _RADIUS ** 2, 0.0, 1.0)
    mbf = maskf.astype(jnp.bfloat16)
    words_ref[0] = jnp.dot(mbf, p_ref[...],
                           preferred_element_type=jnp.float32
                           ).astype(jnp.int32)
    csum = jnp.dot(mbf, pc_ref[...], preferred_element_type=jnp.float32)
    wi = lax.broadcasted_iota(jnp.int32, (_SBLK, _NWORD), 1)
    nw = jnp.min(jnp.where(csum >= float(_K), wi, _NWORD), axis=1,
                 keepdims=True) + 1
    nw_ref[0] = jnp.minimum(nw, _NWORD)
    qb_ref[0] = b1_ref[...] - jnp.dot(nx, w1xt_ref[...],
                                      preferred_element_type=jnp.float32)

    @pl.when(s == 0)
    def _():
        t_ref[0] = (jnp.dot(featT_ref[0], w1ft_ref[...],
                            preferred_element_type=jnp.float32)
                    + jnp.dot(xyz_ref[0], w1xt_ref[...],
                              preferred_element_type=jnp.float32))


def _ballquery_layer1(new_xyz, xyzt, xyz, featT, w1xt, w1ft, b1r, packmat,
                      cummat):
    return pl.pallas_call(
        _bq_body,
        grid=(_B, _S // _SBLK),
        in_specs=[
            pl.BlockSpec((1, _SBLK, 3), lambda b, s: (b, s, 0)),
            pl.BlockSpec((1, 3, _N), lambda b, s: (b, 0, 0)),
            pl.BlockSpec((1, _N, 3), lambda b, s: (b, 0, 0)),
            pl.BlockSpec((1, _N, _C), lambda b, s: (b, 0, 0)),
            pl.BlockSpec((3, _C), lambda b, s: (0, 0)),
            pl.BlockSpec((_C, _C), lambda b, s: (0, 0)),
            pl.BlockSpec((1, _C), lambda b, s: (0, 0)),
            pl.BlockSpec((_N, _NWORD), lambda b, s: (0, 0)),
            pl.BlockSpec((_N, _NWORD), lambda b, s: (0, 0)),
        ],
        out_specs=[
            pl.BlockSpec((1, _SBLK, _NWORD), lambda b, s: (b, s, 0)),
            pl.BlockSpec((1, _SBLK, _C), lambda b, s: (b, s, 0)),
            pl.BlockSpec((1, _N, _C), lambda b, s: (b, 0, 0)),
            pl.BlockSpec((1, _SBLK, 1), lambda b, s: (b, s, 0)),
        ],
        out_shape=[
            jax.ShapeDtypeStruct((_B, _S, _NWORD), jnp.int32),
            jax.ShapeDtypeStruct((_B, _S, _C), jnp.float32),
            jax.ShapeDtypeStruct((_B, _N, _C), jnp.float32),
            jax.ShapeDtypeStruct((_B, _S, 1), jnp.int32),
        ],
    )(new_xyz, xyzt, xyz, featT, w1xt, w1ft, b1r, packmat, cummat)


def _sc_body(words_hbm, t_hbm, nw_hbm, out_hbm,
             words_v, nw_v, idx_v, rows0, rows1, nw_s, fnd_ref,
             semg0, semg1):
    cid = lax.axis_index("c")
    sid = lax.axis_index("s")
    wid = sid * 2 + cid
    g0 = wid * _GPW
    base = (wid // (_NW // _B)) * _N
    pltpu.sync_copy(words_hbm.at[pl.ds(g0, _GPW)], words_v)
    pltpu.sync_copy(nw_hbm.at[pl.ds(g0, _GPW)], nw_v)
    lanes = jnp.arange(16, dtype=jnp.int32)
    lanes_mask = (1 << lanes) - 1

    for g in range(_GPW // 16):
        vec = nw_v[pl.ds(g * 16, 16)]
        for j in range(16):
            nw_s[g * 16 + j] = vec[j]

    def per_centroid(ci, _):
        fnd_ref[0] = 0
        row0 = ci * _K
        ngroups = (nw_s[ci] + 15) >> 4

        def group_step(g, _unused):
            wvec = words_v[ci, pl.ds(g * 16, 16)]
            for j in range(16):
                found = fnd_ref[0]
                w = wvec[j]
                bits = ((w >> lanes) & 1) == 1
                x = w & lanes_mask
                x = x - ((x >> 1) & 0x5555)
                x = (x & 0x3333) + ((x >> 2) & 0x3333)
                x = (x + (x >> 4)) & 0x0F0F
                cntb = (x + (x >> 8)) & 0x1F
                rank = found + cntb
                vals = g * 256 + j * 16 + lanes + base
                pos = jnp.where(bits & (rank < _K), row0 + rank, _TRASH)
                plsc.store_scatter(idx_v, [pos], vals)
                sw = w - ((w >> 1) & 0x5555)
                sw = (sw & 0x3333) + ((sw >> 2) & 0x3333)
                sw = (sw + (sw >> 4)) & 0x0F0F
                fnd_ref[0] = found + ((sw + (sw >> 8)) & 0x1F)
            return 0

        lax.fori_loop(0, ngroups, group_step, 0)
        found = fnd_ref[0]

        first = jnp.full((16,), idx_v[pl.ds(row0, 16)][0], jnp.int32)
        fb = found < _K
        r1 = found + lanes
        r2 = found + 16 + lanes
        plsc.store_scatter(
            idx_v, [jnp.where(fb & (r1 < _K), row0 + r1, _TRASH)], first)
        plsc.store_scatter(
            idx_v, [jnp.where(fb & (r2 < _K), row0 + r2, _TRASH)], first)
        return 0

    lax.fori_loop(0, _GPW, per_centroid, 0)

    nrows = _CB * _K

    def issue(b, rbuf, sem):
        pltpu.async_copy(t_hbm.at[idx_v.at[pl.ds(b * nrows, nrows)]],
                         rbuf, sem)

    def drain(rbuf, sem):
        pltpu.make_async_copy(t_hbm.at[idx_v.at[pl.ds(0, nrows)]],
                              rbuf, sem).wait()

    issue(0, rows0, semg0)
    issue(1, rows1, semg1)

    def batch_step(b2, _u):
        b = 2 * b2
        drain(rows0, semg0)
        pltpu.sync_copy(rows0, out_hbm.at[pl.ds((g0 + b * _CB) * _K, nrows)])
        issue(jnp.minimum(b + 2, _NB - 1), rows0, semg0)
        drain(rows1, semg1)
        pltpu.sync_copy(rows1,
                        out_hbm.at[pl.ds((g0 + (b + 1) * _CB) * _K, nrows)])
        issue(jnp.minimum(b + 3, _NB - 1), rows1, semg1)
        return 0

    lax.fori_loop(0, _NB // 2, batch_step, 0)
    drain(rows0, semg0)
    drain(rows1, semg1)


@functools.cache
def _sc_select_gather():
    return pl.kernel(
        _sc_body,
        out_type=jax.ShapeDtypeStruct((_B * _S * _K, _C), jnp.float32),
        mesh=plsc.VectorSubcoreMesh(core_axis_name="c", subcore_axis_name="s",
                                    num_cores=2, num_subcores=16),
        compiler_params=pltpu.CompilerParams(needs_layout_passes=False,
                                             use_tc_tiling_on_sc=False),
        scratch_types=[
            pltpu.VMEM((_GPW, _NWORD), jnp.int32),
            pltpu.VMEM((_GPW,), jnp.int32),
            pltpu.VMEM((_GPW * _K + 16,), jnp.int32),
            pltpu.VMEM((_CB * _K, _C), jnp.float32),
            pltpu.VMEM((_CB * _K, _C), jnp.float32),
            pltpu.SMEM((_GPW,), jnp.int32),
            pltpu.SMEM((1,), jnp.int32),
            pltpu.SemaphoreType.DMA,
            pltpu.SemaphoreType.DMA,
        ],
    )


def _mlp_body(h_ref, qb_ref, w2t_ref, b2_ref, w3t_ref, b3_ref, o_ref):
    x = h_ref[...]
    q = qb_ref[0]
    qq = jnp.concatenate([q, q], axis=1)
    x3 = x.reshape(_SBLK, _K // 2, 2 * _C)
    h1 = jnp.maximum(x3 + qq[:, None, :], 0.0).reshape(_SBLK * _K // 2,
                                                       2 * _C)
    h2 = jnp.maximum(jnp.dot(h1, w2t_ref[...],
                             preferred_element_type=jnp.float32)
                     + b2_ref[...], 0.0)
    h3 = jnp.maximum(jnp.dot(h2, w3t_ref[...],
                             preferred_element_type=jnp.float32)
                     + b3_ref[...], 0.0)
    m3 = jnp.max(h3.reshape(_SBLK, _K // 2, 256), axis=1)
    o_ref[0] = jnp.maximum(m3[:, :128], m3[:, 128:])


def _mlp_maxpool(h2d, qb, w2t2, b22, w3t2, b32):
    return pl.pallas_call(
        _mlp_body,
        grid=(_B, _S // _SBLK),
        in_specs=[
            pl.BlockSpec((_SBLK * _K // 2, 2 * _C),
                         lambda b, s: (b * (_S // _SBLK) + s, 0)),
            pl.BlockSpec((1, _SBLK, _C), lambda b, s: (b, s, 0)),
            pl.BlockSpec((2 * _C, 2 * _C), lambda b, s: (0, 0)),
            pl.BlockSpec((1, 2 * _C), lambda b, s: (0, 0)),
            pl.BlockSpec((2 * _C, 256), lambda b, s: (0, 0)),
            pl.BlockSpec((1, 256), lambda b, s: (0, 0)),
        ],
        out_specs=pl.BlockSpec((1, _SBLK, 128), lambda b, s: (b, s, 0)),
        out_shape=jax.ShapeDtypeStruct((_B, _S, 128), jnp.float32),
    )(h2d, qb, w2t2, b22, w3t2, b32)


def _packmat():
    n = jnp.arange(_N)
    w = jnp.arange(_NWORD)
    p = jnp.where(n[:, None] // _WPB == w[None, :],
                  2.0 ** (n[:, None] % _WPB), 0.0)
    return p.astype(jnp.bfloat16)


def _cummat():
    n = jnp.arange(_N)
    w = jnp.arange(_NWORD)
    return jnp.where(n[:, None] < (w[None, :] + 1) * _WPB, 1.0,
                     0.0).astype(jnp.bfloat16)


def kernel(xyz, features, W1, b1, W2, b2, W3, b3):
    xyzt3 = jnp.transpose(xyz, (2, 0, 1))
    cxyz = _fps(xyzt3)
    new_xyz = jnp.transpose(cxyz, (2, 1, 0))

    xyzt = jnp.transpose(xyz, (0, 2, 1))
    featT = jnp.transpose(features, (0, 2, 1))
    w1xt = jnp.transpose(W1[:, :3])
    w1ft = jnp.transpose(W1[:, 3:])
    words, qb, t, nw = _ballquery_layer1(
        new_xyz, xyzt, xyz, featT, w1xt, w1ft, b1.reshape(1, _C), _packmat(),
        _cummat())

    h1 = _sc_select_gather()(words.reshape(_B * _S, _NWORD),
                             t.reshape(_B * _N, _C),
                             nw.reshape(_B * _S))
    h2d = h1.reshape(_B * _S * _K // 2, 2 * _C)

    w2t = jnp.transpose(W2)
    w3t = jnp.transpose(W3)
    z = jnp.zeros((_C, _C), jnp.float32)
    z3 = jnp.zeros((_C, 128), jnp.float32)
    w2t2 = jnp.block([[w2t, z], [z, w2t]])
    w3t2 = jnp.block([[w3t, z3], [z3, w3t]])
    b22 = jnp.concatenate([b2, b2]).reshape(1, 2 * _C)
    b32 = jnp.concatenate([b3, b3]).reshape(1, 256)
    feats = _mlp_maxpool(h2d, qb, w2t2, b22, w3t2, b32)
    return new_xyz, jnp.transpose(feats, (0, 2, 1))


# speedup vs baseline: 1.0236x; 1.0236x over previous
import functools

import jax
import jax.numpy as jnp
from jax import lax
from jax.experimental import pallas as pl
from jax.experimental.pallas import tpu as pltpu
from jax.experimental.pallas import tpu_sc as plsc

_B = 8
_N = 4096
_S = 1024
_K = 32
_C = 64
_RADIUS = 0.2
_SBLK = 256
_NW = 32
_GPW = (_B * _S) // _NW
_WPB = 16
_NWORD = _N // _WPB
_CB = 8
_NB = _GPW // _CB
_TRASH = _GPW * _K


def _fps_body(xyzt_ref, cxyz_ref):
    X = xyzt_ref[...]
    lane = lax.broadcasted_iota(jnp.int32, (_B, _N), 1)

    def body(i, carry):
        dist, far = carry
        oh = (lane == far).astype(jnp.float32)
        c = jnp.sum(X * oh[None], axis=2)
        cxyz_ref[:, pl.ds(i, 1), :] = c[:, None, :]
        diff = X - c[:, :, None]
        d = jnp.sum(diff * diff, axis=0)
        dist = jnp.minimum(dist, d)
        m = jnp.max(dist, axis=1, keepdims=True)
        far = jnp.min(jnp.where(dist == m, lane, _N), axis=1,
                      keepdims=True).astype(jnp.int32)
        return dist, far

    dist0 = jnp.full((_B, _N), 1e10, jnp.float32)
    far0 = jnp.zeros((_B, 1), jnp.int32)
    lax.fori_loop(0, _S, body, (dist0, far0))


def _fps(xyzt):
    return pl.pallas_call(
        _fps_body,
        out_shape=jax.ShapeDtypeStruct((3, _S, _B), jnp.float32),
    )(xyzt)


def _bq_body(nx_ref, xyzt_ref, xyz_ref, featT_ref, w1xt_ref, w1ft_ref, b1_ref,
             p_ref, pc_ref, words_ref, qb_ref, t_ref, nw_ref):
    s = pl.program_id(1)
    nx = nx_ref[0]
    xyzt = xyzt_ref[0]
    c2 = jnp.sum(nx * nx, axis=1, keepdims=True)
    p2 = jnp.sum(xyzt * xyzt, axis=0, keepdims=True)
    dot = jnp.dot(nx, xyzt, preferred_element_type=jnp.float32)
    sqr = c2 + p2 - 2.0 * dot
    maskf = jnp.where(sqr > _RADIUS ** 2, 0.0, 1.0)
    mbf = maskf.astype(jnp.bfloat16)
    words_ref[0] = jnp.dot(mbf, p_ref[...],
                           preferred_element_type=jnp.float32
                           ).astype(jnp.int32)
    csum = jnp.dot(mbf, pc_ref[...], preferred_element_type=jnp.float32)
    wi = lax.broadcasted_iota(jnp.int32, (_SBLK, _NWORD), 1)
    nw = jnp.min(jnp.where(csum >= float(_K), wi, _NWORD), axis=1,
                 keepdims=True) + 1
    nw_ref[0] = jnp.minimum(nw, _NWORD)
    qb_ref[0] = b1_ref[...] - jnp.dot(nx, w1xt_ref[...],
                                      preferred_element_type=jnp.float32)

    @pl.when(s == 0)
    def _():
        t_ref[0] = (jnp.dot(featT_ref[0], w1ft_ref[...],
                            preferred_element_type=jnp.float32)
                    + jnp.dot(xyz_ref[0], w1xt_ref[...],
                              preferred_element_type=jnp.float32))


def _ballquery_layer1(new_xyz, xyzt, xyz, featT, w1xt, w1ft, b1r, packmat,
                      cummat):
    return pl.pallas_call(
        _bq_body,
        grid=(_B, _S // _SBLK),
        in_specs=[
            pl.BlockSpec((1, _SBLK, 3), lambda b, s: (b, s, 0)),
            pl.BlockSpec((1, 3, _N), lambda b, s: (b, 0, 0)),
            pl.BlockSpec((1, _N, 3), lambda b, s: (b, 0, 0)),
            pl.BlockSpec((1, _N, _C), lambda b, s: (b, 0, 0)),
            pl.BlockSpec((3, _C), lambda b, s: (0, 0)),
            pl.BlockSpec((_C, _C), lambda b, s: (0, 0)),
            pl.BlockSpec((1, _C), lambda b, s: (0, 0)),
            pl.BlockSpec((_N, _NWORD), lambda b, s: (0, 0)),
            pl.BlockSpec((_N, _NWORD), lambda b, s: (0, 0)),
        ],
        out_specs=[
            pl.BlockSpec((1, _SBLK, _NWORD), lambda b, s: (b, s, 0)),
            pl.BlockSpec((1, _SBLK, _C), lambda b, s: (b, s, 0)),
            pl.BlockSpec((1, _N, _C), lambda b, s: (b, 0, 0)),
            pl.BlockSpec((1, _SBLK, 1), lambda b, s: (b, s, 0)),
        ],
        out_shape=[
            jax.ShapeDtypeStruct((_B, _S, _NWORD), jnp.int32),
            jax.ShapeDtypeStruct((_B, _S, _C), jnp.float32),
            jax.ShapeDtypeStruct((_B, _N, _C), jnp.float32),
            jax.ShapeDtypeStruct((_B, _S, 1), jnp.int32),
        ],
    )(new_xyz, xyzt, xyz, featT, w1xt, w1ft, b1r, packmat, cummat)


def _sc_body(words_hbm, t_hbm, nw_hbm, out_hbm,
             words_v, nw_v, idx_v, rows0, rows1, nw_s, fnd_ref,
             semg0, semg1):
    cid = lax.axis_index("c")
    sid = lax.axis_index("s")
    wid = sid * 2 + cid
    g0 = wid * _GPW
    base = (wid // (_NW // _B)) * _N
    pltpu.sync_copy(words_hbm.at[pl.ds(g0, _GPW)], words_v)
    pltpu.sync_copy(nw_hbm.at[pl.ds(g0, _GPW)], nw_v)
    lanes = jnp.arange(16, dtype=jnp.int32)
    lanes_mask = (1 << lanes) - 1

    for g in range(_GPW // 16):
        vec = nw_v[pl.ds(g * 16, 16)]
        for j in range(16):
            nw_s[g * 16 + j] = vec[j]

    def per_centroid(ci, _):
        fnd_ref[0] = 0
        row0 = ci * _K
        ngroups = (nw_s[ci] + 15) >> 4

        def group_step(g, _unused):
            wvec = words_v[ci, pl.ds(g * 16, 16)]
            for j in range(16):
                found = fnd_ref[0]
                w = wvec[j]
                bits = ((w >> lanes) & 1) == 1
                x = w & lanes_mask
                x = x - ((x >> 1) & 0x5555)
                x = (x & 0x3333) + ((x >> 2) & 0x3333)
                x = (x + (x >> 4)) & 0x0F0F
                cntb = (x + (x >> 8)) & 0x1F
                rank = found + cntb
                vals = g * 256 + j * 16 + lanes + base
                pos = jnp.where(bits & (rank < _K), row0 + rank, _TRASH)
                plsc.store_scatter(idx_v, [pos], vals)
                sw = w - ((w >> 1) & 0x5555)
                sw = (sw & 0x3333) + ((sw >> 2) & 0x3333)
                sw = (sw + (sw >> 4)) & 0x0F0F
                fnd_ref[0] = found + ((sw + (sw >> 8)) & 0x1F)
            return 0

        lax.fori_loop(0, ngroups, group_step, 0)
        found = fnd_ref[0]

        first = jnp.full((16,), idx_v[pl.ds(row0, 16)][0], jnp.int32)
        fb = found < _K
        r1 = found + lanes
        r2 = found + 16 + lanes
        plsc.store_scatter(
            idx_v, [jnp.where(fb & (r1 < _K), row0 + r1, _TRASH)], first)
        plsc.store_scatter(
            idx_v, [jnp.where(fb & (r2 < _K), row0 + r2, _TRASH)], first)
        return 0

    lax.fori_loop(0, _GPW, per_centroid, 0)

    nrows = _CB * _K

    def issue(b, rbuf, sem):
        pltpu.async_copy(t_hbm.at[idx_v.at[pl.ds(b * nrows, nrows)]],
                         rbuf, sem)

    def drain(rbuf, sem):
        pltpu.make_async_copy(t_hbm.at[idx_v.at[pl.ds(0, nrows)]],
                              rbuf, sem).wait()

    issue(0, rows0, semg0)
    issue(1, rows1, semg1)

    def batch_step(b2, _u):
        b = 2 * b2
        drain(rows0, semg0)
        pltpu.sync_copy(rows0, out_hbm.at[pl.ds((g0 + b * _CB) * _K, nrows)])
        issue(jnp.minimum(b + 2, _NB - 1), rows0, semg0)
        drain(rows1, semg1)
        pltpu.sync_copy(rows1,
                        out_hbm.at[pl.ds((g0 + (b + 1) * _CB) * _K, nrows)])
        issue(jnp.minimum(b + 3, _NB - 1), rows1, semg1)
        return 0

    lax.fori_loop(0, _NB // 2, batch_step, 0)
    drain(rows0, semg0)
    drain(rows1, semg1)


@functools.cache
def _sc_select_gather():
    return pl.kernel(
        _sc_body,
        out_type=jax.ShapeDtypeStruct((_B * _S * _K, _C), jnp.float32),
        mesh=plsc.VectorSubcoreMesh(core_axis_name="c", subcore_axis_name="s",
                                    num_cores=2, num_subcores=16),
        compiler_params=pltpu.CompilerParams(needs_layout_passes=False,
                                             use_tc_tiling_on_sc=False),
        scratch_types=[
            pltpu.VMEM((_GPW, _NWORD), jnp.int32),
            pltpu.VMEM((_GPW,), jnp.int32),
            pltpu.VMEM((_GPW * _K + 16,), jnp.int32),
            pltpu.VMEM((_CB * _K, _C), jnp.float32),
            pltpu.VMEM((_CB * _K, _C), jnp.float32),
            pltpu.SMEM((_GPW,), jnp.int32),
            pltpu.SMEM((1,), jnp.int32),
            pltpu.SemaphoreType.DMA,
            pltpu.SemaphoreType.DMA,
        ],
    )


def _mlp_body(h_ref, qb_ref, w2t_ref, b2_ref, w3t_ref, b3_ref, o_ref):
    x = h_ref[...]
    q = qb_ref[0]
    qq = jnp.concatenate([q, q], axis=1)
    x3 = x.reshape(_SBLK, _K // 2, 2 * _C)
    h1 = jnp.maximum(x3 + qq[:, None, :], 0.0).reshape(_SBLK * _K // 2,
                                                       2 * _C)
    h2 = jnp.maximum(jnp.dot(h1, w2t_ref[...],
                             preferred_element_type=jnp.float32)
                     + b2_ref[...], 0.0)
    h3 = jnp.maximum(jnp.dot(h2, w3t_ref[...],
                             preferred_element_type=jnp.float32)
                     + b3_ref[...], 0.0)
    m3 = jnp.max(h3.reshape(_SBLK, _K // 2, 256), axis=1)
    o_ref[0] = jnp.maximum(m3[:, :128], m3[:, 128:])


def _mlp_maxpool(h2d, qb, w2t2, b22, w3t2, b32):
    return pl.pallas_call(
        _mlp_body,
        grid=(_B, _S // _SBLK),
        in_specs=[
            pl.BlockSpec((_SBLK * _K // 2, 2 * _C),
                         lambda b, s: (b * (_S // _SBLK) + s, 0)),
            pl.BlockSpec((1, _SBLK, _C), lambda b, s: (b, s, 0)),
            pl.BlockSpec((2 * _C, 2 * _C), lambda b, s: (0, 0)),
            pl.BlockSpec((1, 2 * _C), lambda b, s: (0, 0)),
            pl.BlockSpec((2 * _C, 256), lambda b, s: (0, 0)),
            pl.BlockSpec((1, 256), lambda b, s: (0, 0)),
        ],
        out_specs=pl.BlockSpec((1, _SBLK, 128), lambda b, s: (b, s, 0)),
        out_shape=jax.ShapeDtypeStruct((_B, _S, 128), jnp.float32),
    )(h2d, qb, w2t2, b22, w3t2, b32)


def _packmat():
    n = jnp.arange(_N)
    w = jnp.arange(_NWORD)
    p = jnp.where(n[:, None] // _WPB == w[None, :],
                  2.0 ** (n[:, None] % _WPB), 0.0)
    return p.astype(jnp.bfloat16)


def _cummat():
    n = jnp.arange(_N)
    w = jnp.arange(_NWORD)
    return jnp.where(n[:, None] < (w[None, :] + 1) * _WPB, 1.0,
                     0.0).astype(jnp.bfloat16)


def kernel(xyz, features, W1, b1, W2, b2, W3, b3):
    xyzt3 = jnp.transpose(xyz, (2, 0, 1))
    cxyz = _fps(xyzt3)
    new_xyz = jnp.transpose(cxyz, (2, 1, 0))

    xyzt = jnp.transpose(xyz, (0, 2, 1))
    featT = jnp.transpose(features, (0, 2, 1))
    w1xt = jnp.transpose(W1[:, :3])
    w1ft = jnp.transpose(W1[:, 3:])
    words, qb, t, nw = _ballquery_layer1(
        new_xyz, xyzt, xyz, featT, w1xt, w1ft, b1.reshape(1, _C), _packmat(),
        _cummat())

    h1 = _sc_select_gather()(words.reshape(_B * _S, _NWORD),
                             t.reshape(_B * _N, _C),
                             nw.reshape(_B * _S))
    h2d = h1.reshape(_B * _S * _K // 2, 2 * _C)

    w2t = jnp.transpose(W2)
    w3t = jnp.transpose(W3)
    z = jnp.zeros((_C, _C), jnp.float32)
    z3 = jnp.zeros((_C, 128), jnp.float32)
    w2t2 = jnp.block([[w2t, z], [z, w2t]])
    w3t2 = jnp.block([[w3t, z3], [z3, w3t]])
    b22 = jnp.concatenate([b2, b2]).reshape(1, 2 * _C)
    b32 = jnp.concatenate([b3, b3]).reshape(1, 256)
    feats = _mlp_maxpool(h2d, qb, w2t2, b22, w3t2, b32)
    return new_xyz, jnp.transpose(feats, (0, 2, 1))
